# baseline (device time: 108917 ns/iter reference)
import jax
import jax.numpy as jnp
from jax import lax
from jax.experimental import pallas as pl
from jax.experimental.pallas import tpu as pltpu

N_DEV = 8
SQ = 1024
D = 1024
HQ = 8
DH = 128
BLK = 64
SCALE = 0.08838834764831843
N_CHUNKS = 8
CHUNK = SQ // N_CHUNKS


def kernel(x, Wq, K_ext, V_ext, Wo):
    x2 = x.reshape(SQ, D)
    k2 = K_ext.reshape(SQ, HQ * DH)
    v2 = V_ext.reshape(SQ, HQ * DH)

    def body(x_ref, wq_ref, k_ref, v_ref, wo_ref, out_ref,
             entry_left, entry_right, send_sems, recv_sems):
        my = lax.axis_index("i")
        left = (my - 1) % N_DEV
        right = (my + 1) % N_DEV
        is_first = my == 0
        is_last = my == N_DEV - 1

        @pl.when(jnp.logical_not(is_first))
        def _():
            pl.semaphore_signal(entry_right, inc=1, device_id=(left,),
                                device_id_type=pl.DeviceIdType.MESH)

        @pl.when(jnp.logical_not(is_last))
        def _():
            pl.semaphore_signal(entry_left, inc=1, device_id=(right,),
                                device_id_type=pl.DeviceIdType.MESH)

        @pl.when(jnp.logical_not(is_first))
        def _():
            pl.semaphore_wait(entry_left, 1)

        @pl.when(jnp.logical_not(is_last))
        def _():
            pl.semaphore_wait(entry_right, 1)

        @pl.when(is_first)
        def _compute():
            q = jnp.dot(x_ref[...], wq_ref[...],
                        preferred_element_type=jnp.float32)
            qb = lax.broadcasted_iota(jnp.int32, (SQ, SQ), 0) // BLK
            kb = lax.broadcasted_iota(jnp.int32, (SQ, SQ), 1) // BLK
            mask = kb <= qb
            for h in range(HQ):
                qh = q[:, h * DH:(h + 1) * DH]
                kh = k_ref[:, h * DH:(h + 1) * DH]
                vh = v_ref[:, h * DH:(h + 1) * DH]
                s = lax.dot_general(qh, kh, (((1,), (1,)), ((), ())),
                                    preferred_element_type=jnp.float32)
                s = jnp.where(mask, s * SCALE, -1e9)
                m = jnp.max(s, axis=1, keepdims=True)
                w = jnp.exp(s - m)
                w = w / jnp.sum(w, axis=1, keepdims=True)
                ctx = jnp.dot(w, vh, preferred_element_type=jnp.float32)
                part = jnp.dot(ctx, wo_ref[h * DH:(h + 1) * DH, :],
                               preferred_element_type=jnp.float32)
                if h == 0:
                    out_ref[...] = part
                else:
                    out_ref[...] += part

        def chunk_desc(c):
            sl = pl.ds(c * CHUNK, CHUNK)
            return pltpu.make_async_remote_copy(
                src_ref=out_ref.at[sl, :],
                dst_ref=out_ref.at[sl, :],
                send_sem=send_sems.at[c],
                recv_sem=recv_sems.at[c],
                device_id=(right,),
                device_id_type=pl.DeviceIdType.MESH,
            )

        for c in range(N_CHUNKS):
            desc = chunk_desc(c)

            @pl.when(jnp.logical_not(is_first))
            def _(desc=desc):
                desc.wait_recv()

            @pl.when(jnp.logical_not(is_last))
            def _(desc=desc):
                desc.start()

        for c in range(N_CHUNKS):
            desc = chunk_desc(c)

            @pl.when(jnp.logical_not(is_last))
            def _(desc=desc):
                desc.wait_send()

    out = pl.pallas_call(
        body,
        out_shape=jax.ShapeDtypeStruct((SQ, D), jnp.float32),
        in_specs=[pl.BlockSpec(memory_space=pltpu.VMEM)] * 5,
        out_specs=pl.BlockSpec(memory_space=pltpu.VMEM),
        scratch_shapes=[
            pltpu.SemaphoreType.REGULAR,
            pltpu.SemaphoreType.REGULAR,
            pltpu.SemaphoreType.DMA((N_CHUNKS,)),
            pltpu.SemaphoreType.DMA((N_CHUNKS,)),
        ],
    )(x2, Wq, k2, v2, Wo)
    return out.reshape(1, SQ, D)


# device time: 81669 ns/iter; 1.3336x vs baseline; 1.3336x over previous
import jax
import jax.numpy as jnp
from jax import lax
from jax.experimental import pallas as pl
from jax.experimental.pallas import tpu as pltpu

N_DEV = 8
SQ = 1024
D = 1024
HQ = 8
DH = 128
BLK = 64
SCALE = 0.08838834764831843
N_CHUNKS = 8
CHUNK = SQ // N_CHUNKS


def kernel(x, Wq, K_ext, V_ext, Wo):
    x2 = x.reshape(SQ, D)
    k2 = K_ext.reshape(SQ, HQ * DH)
    v2 = V_ext.reshape(SQ, HQ * DH)

    def body(x_ref, wq_ref, k_ref, v_ref, wo_ref, out_ref,
             ready_sem, send_sems, far_send_sems, recv_sems):
        my = lax.axis_index("i")
        nxt = (my + 1) % N_DEV
        prev = jnp.where(my == 4, 0, (my - 1) % N_DEV)
        is_src = my == 0
        is_tail = jnp.logical_or(my == 3, my == N_DEV - 1)
        has_recv = my != 0
        fwds = jnp.logical_not(is_tail)

        @pl.when(has_recv)
        def _():
            pl.semaphore_signal(ready_sem, inc=1, device_id=(prev,),
                                device_id_type=pl.DeviceIdType.MESH)

        @pl.when(is_src)
        def _():
            pl.semaphore_wait(ready_sem, 2)

        @pl.when(jnp.logical_and(has_recv, fwds))
        def _():
            pl.semaphore_wait(ready_sem, 1)

        def compute_chunk(c):
            L = (c + 1) * CHUNK
            qx = x_ref[pl.ds(c * CHUNK, CHUNK), :]
            q = jnp.dot(qx, wq_ref[...], preferred_element_type=jnp.float32)
            ri = lax.broadcasted_iota(jnp.int32, (CHUNK, L), 0)
            ci = lax.broadcasted_iota(jnp.int32, (CHUNK, L), 1)
            mask = (ci // BLK) <= (ri // BLK + 2 * c)
            for h in range(HQ):
                hs = slice(h * DH, (h + 1) * DH)
                qh = q[:, hs]
                kh = k_ref[0:L, hs]
                vh = v_ref[0:L, hs]
                s = lax.dot_general(qh, kh, (((1,), (1,)), ((), ())),
                                    preferred_element_type=jnp.float32)
                s = jnp.where(mask, s * SCALE, -1e9)
                m = jnp.max(s, axis=1, keepdims=True)
                w = jnp.exp(s - m)
                w = w / jnp.sum(w, axis=1, keepdims=True)
                ctx = jnp.dot(w, vh, preferred_element_type=jnp.float32)
                part = jnp.dot(ctx, wo_ref[hs, :],
                               preferred_element_type=jnp.float32)
                if h == 0:
                    out_ref[pl.ds(c * CHUNK, CHUNK), :] = part
                else:
                    out_ref[pl.ds(c * CHUNK, CHUNK), :] += part

        def chunk_desc(c, sems, target):
            sl = pl.ds(c * CHUNK, CHUNK)
            return pltpu.make_async_remote_copy(
                src_ref=out_ref.at[sl, :],
                dst_ref=out_ref.at[sl, :],
                send_sem=sems.at[c],
                recv_sem=recv_sems.at[c],
                device_id=(target,),
                device_id_type=pl.DeviceIdType.MESH,
            )

        for c in range(N_CHUNKS):
            @pl.when(is_src)
            def _(c=c):
                compute_chunk(c)

            desc = chunk_desc(c, send_sems, nxt)

            @pl.when(has_recv)
            def _(desc=desc):
                desc.wait_recv()

            @pl.when(fwds)
            def _(desc=desc):
                desc.start()

            @pl.when(is_src)
            def _(c=c):
                chunk_desc(c, far_send_sems, 4).start()

        for c in range(N_CHUNKS):
            @pl.when(fwds)
            def _(c=c):
                chunk_desc(c, send_sems, nxt).wait_send()

            @pl.when(is_src)
            def _(c=c):
                chunk_desc(c, far_send_sems, 4).wait_send()

    out = pl.pallas_call(
        body,
        out_shape=jax.ShapeDtypeStruct((SQ, D), jnp.float32),
        in_specs=[pl.BlockSpec(memory_space=pltpu.VMEM)] * 5,
        out_specs=pl.BlockSpec(memory_space=pltpu.VMEM),
        scratch_shapes=[
            pltpu.SemaphoreType.REGULAR,
            pltpu.SemaphoreType.DMA((N_CHUNKS,)),
            pltpu.SemaphoreType.DMA((N_CHUNKS,)),
            pltpu.SemaphoreType.DMA((N_CHUNKS,)),
        ],
    )(x2, Wq, k2, v2, Wo)
    return out.reshape(1, SQ, D)


# device time: 78434 ns/iter; 1.3886x vs baseline; 1.0412x over previous
import jax
import jax.numpy as jnp
from jax import lax
from jax.experimental import pallas as pl
from jax.experimental.pallas import tpu as pltpu

N_DEV = 8
SQ = 1024
D = 1024
HQ = 8
DH = 128
BLK = 64
SCALE = 0.08838834764831843
N_CHUNKS = 8
CHUNK = SQ // N_CHUNKS

_MESH = pl.DeviceIdType.MESH


def kernel(x, Wq, K_ext, V_ext, Wo):
    def body(x_ref, wq_ref, k_ref, v_ref, wo_ref, out_ref, ctx_ref,
             ready_a, ready_b, ready_c, send_sems, far_b_sems, far_c_sems,
             recv_sems):
        my = lax.axis_index("i")
        is_src = my == 0
        prev = jnp.where(my == 2, 1,
               jnp.where(my == 5, 4,
               jnp.where(my == 6, 5,
               jnp.where(my == 7, 3, 0))))
        nxt = jnp.where(my == 1, 2,
              jnp.where(my == 3, 7,
              jnp.where(my == 4, 5,
              jnp.where(my == 5, 6, 0))))
        is_fwd = jnp.logical_or(
            jnp.logical_or(my == 1, my == 3),
            jnp.logical_or(my == 4, my == 5))
        has_recv = my != 0

        @pl.when(my == 3)
        def _():
            pl.semaphore_signal(ready_b, inc=1, device_id=(0,),
                                device_id_type=_MESH)

        @pl.when(my == 4)
        def _():
            pl.semaphore_signal(ready_c, inc=1, device_id=(0,),
                                device_id_type=_MESH)

        @pl.when(jnp.logical_and(has_recv,
                                 jnp.logical_and(my != 3, my != 4)))
        def _():
            pl.semaphore_signal(ready_a, inc=1, device_id=(prev,),
                                device_id_type=_MESH)

        @pl.when(is_src)
        def _():
            pl.semaphore_wait(ready_a, 1)
            pl.semaphore_wait(ready_b, 1)
            pl.semaphore_wait(ready_c, 1)

        @pl.when(is_fwd)
        def _():
            pl.semaphore_wait(ready_a, 1)

        def compute_ctx_chunk(c):
            L = (c + 1) * CHUNK
            sl = pl.ds(c * CHUNK, CHUNK)
            qx = x_ref[0, sl, :]
            q = jnp.dot(qx, wq_ref[...], preferred_element_type=jnp.float32)
            ri = lax.broadcasted_iota(jnp.int32, (CHUNK, L), 0)
            ci = lax.broadcasted_iota(jnp.int32, (CHUNK, L), 1)
            mask = (ci // BLK) <= (ri // BLK + 2 * c)
            for h in range(HQ):
                qh = q[:, h * DH:(h + 1) * DH]
                kh = k_ref[0, 0:L, h, :]
                vh = v_ref[0, 0:L, h, :]
                s = lax.dot_general(qh, kh, (((1,), (1,)), ((), ())),
                                    preferred_element_type=jnp.float32)
                s = jnp.where(mask, s * SCALE, -1e9)
                m = jnp.max(s, axis=1, keepdims=True)
                w = jnp.exp(s - m)
                w = w / jnp.sum(w, axis=1, keepdims=True)
                ctx_ref[sl, h * DH:(h + 1) * DH] = jnp.dot(
                    w, vh, preferred_element_type=jnp.float32)

        def chunk_desc(c, sems, target):
            sl = pl.ds(c * CHUNK, CHUNK)
            return pltpu.make_async_remote_copy(
                src_ref=ctx_ref.at[sl, :],
                dst_ref=ctx_ref.at[sl, :],
                send_sem=sems.at[c],
                recv_sem=recv_sems.at[c],
                device_id=(target,),
                device_id_type=_MESH,
            )

        def wo_chunk(c):
            sl = pl.ds(c * CHUNK, CHUNK)
            out_ref[0, sl, :] = jnp.dot(ctx_ref[sl, :], wo_ref[...],
                                        preferred_element_type=jnp.float32)

        for c in range(N_CHUNKS):
            @pl.when(is_src)
            def _(c=c):
                compute_ctx_chunk(c)
                chunk_desc(c, send_sems, 1).start()
                chunk_desc(c, far_b_sems, 3).start()
                chunk_desc(c, far_c_sems, 4).start()

            @pl.when(has_recv)
            def _(c=c):
                chunk_desc(c, send_sems, nxt).wait_recv()

            @pl.when(is_fwd)
            def _(c=c):
                chunk_desc(c, send_sems, nxt).start()

            @pl.when(has_recv)
            def _(c=c):
                wo_chunk(c)

        @pl.when(is_src)
        def _():
            for c in range(N_CHUNKS):
                wo_chunk(c)

        for c in range(N_CHUNKS):
            @pl.when(is_src)
            def _(c=c):
                chunk_desc(c, send_sems, 1).wait_send()
                chunk_desc(c, far_b_sems, 3).wait_send()
                chunk_desc(c, far_c_sems, 4).wait_send()

            @pl.when(is_fwd)
            def _(c=c):
                chunk_desc(c, send_sems, nxt).wait_send()

    out = pl.pallas_call(
        body,
        out_shape=jax.ShapeDtypeStruct((1, SQ, D), jnp.float32),
        in_specs=[pl.BlockSpec(memory_space=pltpu.VMEM)] * 5,
        out_specs=pl.BlockSpec(memory_space=pltpu.VMEM),
        scratch_shapes=[
            pltpu.VMEM((SQ, D), jnp.float32),
            pltpu.SemaphoreType.REGULAR,
            pltpu.SemaphoreType.REGULAR,
            pltpu.SemaphoreType.REGULAR,
            pltpu.SemaphoreType.DMA((N_CHUNKS,)),
            pltpu.SemaphoreType.DMA((N_CHUNKS,)),
            pltpu.SemaphoreType.DMA((N_CHUNKS,)),
            pltpu.SemaphoreType.DMA((N_CHUNKS,)),
        ],
    )(x, Wq, K_ext, V_ext, Wo)
    return out


# device time: 59666 ns/iter; 1.8254x vs baseline; 1.3146x over previous
import jax
import jax.numpy as jnp
from jax import lax
from jax.experimental import pallas as pl
from jax.experimental.pallas import tpu as pltpu

SQ = 1024
D = 1024
HQ = 8
DH = 128
BLK = 64
SCALE = 0.08838834764831843
N_CHUNKS = 8
CHUNK = SQ // N_CHUNKS


def kernel(x, Wq, K_ext, V_ext, Wo):
    def body(x_ref, wq_ref, k_ref, v_ref, wo_ref, out_ref, ctx_ref):
        for c in range(N_CHUNKS):
            L = (c + 1) * CHUNK
            sl = pl.ds(c * CHUNK, CHUNK)
            qx = x_ref[0, sl, :]
            q = jnp.dot(qx, wq_ref[...], preferred_element_type=jnp.float32)
            ri = lax.broadcasted_iota(jnp.int32, (CHUNK, L), 0)
            ci = lax.broadcasted_iota(jnp.int32, (CHUNK, L), 1)
            mask = (ci // BLK) <= (ri // BLK + 2 * c)
            for h in range(HQ):
                qh = q[:, h * DH:(h + 1) * DH]
                kh = k_ref[0, 0:L, h, :]
                vh = v_ref[0, 0:L, h, :]
                s = lax.dot_general(qh, kh, (((1,), (1,)), ((), ())),
                                    preferred_element_type=jnp.float32)
                s = jnp.where(mask, s * SCALE, -1e9)
                m = jnp.max(s, axis=1, keepdims=True)
                w = jnp.exp(s - m)
                w = w / jnp.sum(w, axis=1, keepdims=True)
                ctx_ref[sl, h * DH:(h + 1) * DH] = jnp.dot(
                    w, vh, preferred_element_type=jnp.float32)
        for c in range(N_CHUNKS):
            sl = pl.ds(c * CHUNK, CHUNK)
            out_ref[0, sl, :] = jnp.dot(ctx_ref[sl, :], wo_ref[...],
                                        preferred_element_type=jnp.float32)

    return pl.pallas_call(
        body,
        out_shape=jax.ShapeDtypeStruct((1, SQ, D), jnp.float32),
        in_specs=[pl.BlockSpec(memory_space=pltpu.VMEM)] * 5,
        out_specs=pl.BlockSpec(memory_space=pltpu.VMEM),
        scratch_shapes=[pltpu.VMEM((SQ, D), jnp.float32)],
    )(x, Wq, K_ext, V_ext, Wo)
